# baseline (speedup 1.0000x reference)
import jax
import jax.numpy as jnp
from jax.experimental import pallas as pl
from jax.experimental.pallas import tpu as pltpu

_BM = 500


def _gcn_body(x_ref, w_ref, b_ref, adj_ref, out_ref, sup_ref):
    @pl.when(pl.program_id(0) == 0)
    def _():
        sup_ref[...] = jnp.dot(
            x_ref[...], w_ref[...], preferred_element_type=jnp.float32
        )

    out_ref[0] = (
        jnp.dot(adj_ref[0], sup_ref[...], preferred_element_type=jnp.float32)
        + b_ref[...]
    )


def kernel(input, adj, weight, bias):
    n, in_f = input.shape
    out_f = weight.shape[1]
    bm = _BM if n % _BM == 0 else n
    nb = n // bm
    bias2d = bias.reshape(1, out_f)
    adj3 = adj.reshape(nb, bm, n)
    out3 = pl.pallas_call(
        _gcn_body,
        grid=(nb,),
        in_specs=[
            pl.BlockSpec((n, in_f), lambda i: (0, 0)),
            pl.BlockSpec((in_f, out_f), lambda i: (0, 0)),
            pl.BlockSpec((1, out_f), lambda i: (0, 0)),
            pl.BlockSpec((1, bm, n), lambda i: (i, 0, 0)),
        ],
        out_specs=pl.BlockSpec((1, bm, out_f), lambda i: (i, 0, 0)),
        out_shape=jax.ShapeDtypeStruct((nb, bm, out_f), jnp.float32),
        scratch_shapes=[pltpu.VMEM((n, out_f), jnp.float32)],
        compiler_params=pltpu.CompilerParams(vmem_limit_bytes=64 * 1024 * 1024),
    )(input, weight, bias2d, adj3)
    return out3.reshape(n, out_f)


# speedup vs baseline: 3.4547x; 3.4547x over previous
import jax
import jax.numpy as jnp
from jax.experimental import pallas as pl
from jax.experimental.pallas import tpu as pltpu

_BM = 200
_S = 4


def _make_body(bm, s_slots, n):
    def _gcn_body(x_ref, w_ref, b_ref, adj_hbm, out_ref, sup_ref, ring, sems):
        i = pl.program_id(0)
        nb = pl.num_programs(0)

        @pl.when(i == 0)
        def _():
            for s in range(s_slots):
                pltpu.make_async_copy(
                    adj_hbm.at[pl.ds(s * bm, bm), :], ring.at[s], sems.at[s]
                ).start()
            sup_ref[...] = jnp.dot(
                x_ref[...], w_ref[...], preferred_element_type=jnp.float32
            )

        @pl.when(jnp.logical_and(i > 0, i - 1 + s_slots < nb))
        def _():
            j = i - 1 + s_slots
            slot = jax.lax.rem(i - 1, s_slots)
            pltpu.make_async_copy(
                adj_hbm.at[pl.ds(j * bm, bm), :], ring.at[slot], sems.at[slot]
            ).start()

        slot = jax.lax.rem(i, s_slots)
        pltpu.make_async_copy(
            adj_hbm.at[pl.ds(i * bm, bm), :], ring.at[slot], sems.at[slot]
        ).wait()
        out_ref[...] = (
            jnp.dot(ring[slot], sup_ref[...], preferred_element_type=jnp.float32)
            + b_ref[...]
        )

    return _gcn_body


def kernel(input, adj, weight, bias):
    n, in_f = input.shape
    out_f = weight.shape[1]
    bm = _BM if n % _BM == 0 else n
    nb = n // bm
    s_slots = min(_S, nb)
    bias2d = bias.reshape(1, out_f)
    return pl.pallas_call(
        _make_body(bm, s_slots, n),
        grid=(nb,),
        in_specs=[
            pl.BlockSpec((n, in_f), lambda i: (0, 0)),
            pl.BlockSpec((in_f, out_f), lambda i: (0, 0)),
            pl.BlockSpec((1, out_f), lambda i: (0, 0)),
            pl.BlockSpec(memory_space=pltpu.MemorySpace.HBM),
        ],
        out_specs=pl.BlockSpec((bm, out_f), lambda i: (i, 0)),
        out_shape=jax.ShapeDtypeStruct((n, out_f), jnp.float32),
        scratch_shapes=[
            pltpu.VMEM((n, out_f), jnp.float32),
            pltpu.VMEM((s_slots, bm, n), jnp.float32),
            pltpu.SemaphoreType.DMA((s_slots,)),
        ],
    )(input, weight, bias2d, adj)
